# baseline (speedup 1.0000x reference)
import functools

import jax
import jax.numpy as jnp
from jax import lax
from jax.experimental import pallas as pl
from jax.experimental.pallas import tpu as pltpu
from jax.experimental.pallas import tpu_sc as plsc

N_USERS = 1000000
N_ITEMS = 100000
N_FEATS = 100
K = 32
B = 4096

NC = 2
NS = 16
NW = NC * NS
BPW = B // NW

B_TILE = 256
D = K + 1
VF = (2 + N_FEATS) * K


def _sc_gather_body(u_hbm, i_hbm, ut_hbm, it_hbm, out_u, out_i,
                    uidx_v, urows_v, iidx_v, irows_v, sem_u, sem_i):
  wid = lax.axis_index("s") * NC + lax.axis_index("c")
  base = wid * BPW
  pltpu.sync_copy(u_hbm.at[pl.ds(base, BPW)], uidx_v)
  pltpu.sync_copy(i_hbm.at[pl.ds(base, BPW)], iidx_v)

  def fire(g, carry):
    uch = uidx_v[pl.ds(g * 16, 16)]
    ich = iidx_v[pl.ds(g * 16, 16)]
    for t in range(16):
      pltpu.make_async_copy(ut_hbm.at[pl.ds(uch[t], 1)],
                            urows_v.at[pl.ds(g * 16 + t, 1)], sem_u).start()
      pltpu.make_async_copy(it_hbm.at[pl.ds(ich[t], 1)],
                            irows_v.at[pl.ds(g * 16 + t, 1)], sem_i).start()
    return carry

  lax.fori_loop(0, BPW // 16, fire, 0)
  pltpu.make_async_copy(ut_hbm.at[pl.ds(0, BPW)], urows_v, sem_u).wait()
  pltpu.make_async_copy(it_hbm.at[pl.ds(0, BPW)], irows_v, sem_i).wait()

  pltpu.sync_copy(urows_v, out_u.at[pl.ds(base, BPW)])
  pltpu.sync_copy(irows_v, out_i.at[pl.ds(base, BPW)])


@functools.cache
def _sc_gather():
  return pl.kernel(
      _sc_gather_body,
      mesh=plsc.VectorSubcoreMesh(core_axis_name="c", subcore_axis_name="s"),
      out_type=(
          jax.ShapeDtypeStruct((B, D), jnp.float32),
          jax.ShapeDtypeStruct((B, D), jnp.float32),
      ),
      scratch_types=[
          pltpu.VMEM((BPW,), jnp.int32),
          pltpu.VMEM((BPW, D), jnp.float32),
          pltpu.VMEM((BPW,), jnp.int32),
          pltpu.VMEM((BPW, D), jnp.float32),
          pltpu.SemaphoreType.DMA,
          pltpu.SemaphoreType.DMA,
      ],
  )


def _tc_body(w0_ref, feats_ref, uv_ref, iv_ref, ftab_ref, ftab3_ref,
             s_ref, w_ref, v_ref):
  feats = feats_ref[...]
  uv = uv_ref[...]
  iv = iv_ref[...]
  uvk = uv[:, :K]
  ivk = iv[:, :K]
  ftab_k = ftab_ref[:, :K]
  ftab_w = ftab_ref[:, K:]

  v_ref[:, 0:1, :] = uvk[:, None, :]
  v_ref[:, 1:2, :] = ivk[:, None, :]
  v_ref[:, 2:, :] = feats[:, :, None] * ftab3_ref[...]

  wfeat = feats * ftab_w.reshape(1, N_FEATS)
  w_ref[:, 0:1] = uv[:, K:]
  w_ref[:, 1:2] = iv[:, K:]
  w_ref[:, 2:] = wfeat

  sv = uvk + ivk + jnp.dot(feats, ftab_k, preferred_element_type=jnp.float32)
  sq = (uvk * uvk + ivk * ivk
        + jnp.dot(feats * feats, ftab_k * ftab_k,
                  preferred_element_type=jnp.float32))
  v_ = 0.5 * (jnp.sum(sv * sv, axis=1, keepdims=True)
              - jnp.sum(sq, axis=1, keepdims=True))
  w_sum = (uv[:, K:] + iv[:, K:]
           + jnp.sum(wfeat, axis=1, keepdims=True))
  s_ref[...] = w0_ref[0, 0] + w_sum + v_


def _tc_fm(w0, feats, uv, iv, ftab, ftab3):
  grid = (B // B_TILE,)
  return pl.pallas_call(
      _tc_body,
      grid=grid,
      in_specs=[
          pl.BlockSpec(memory_space=pltpu.SMEM),
          pl.BlockSpec((B_TILE, N_FEATS), lambda b: (b, 0)),
          pl.BlockSpec((B_TILE, D), lambda b: (b, 0)),
          pl.BlockSpec((B_TILE, D), lambda b: (b, 0)),
          pl.BlockSpec((N_FEATS, D), lambda b: (0, 0)),
          pl.BlockSpec((1, N_FEATS, K), lambda b: (0, 0, 0)),
      ],
      out_specs=[
          pl.BlockSpec((B_TILE, 1), lambda b: (b, 0)),
          pl.BlockSpec((B_TILE, 2 + N_FEATS), lambda b: (b, 0)),
          pl.BlockSpec((B_TILE, 2 + N_FEATS, K), lambda b: (b, 0, 0)),
      ],
      out_shape=(
          jax.ShapeDtypeStruct((B, 1), jnp.float32),
          jax.ShapeDtypeStruct((B, 2 + N_FEATS), jnp.float32),
          jax.ShapeDtypeStruct((B, 2 + N_FEATS, K), jnp.float32),
      ),
  )(w0, feats, uv, iv, ftab, ftab3)


def kernel(u, i, feats, user_table, item_table, feat_table, w0):
  u1 = u.reshape(B).astype(jnp.int32)
  i1 = i.reshape(B).astype(jnp.int32)

  uv, iv = _sc_gather()(u1, i1, user_table, item_table)

  ftab3 = feat_table[:, :K][None]
  s, w, v = _tc_fm(w0.reshape(1, 1), feats, uv, iv, feat_table, ftab3)
  return (s.reshape(B), w, v)


# speedup vs baseline: 4.9096x; 4.9096x over previous
import functools

import jax
import jax.numpy as jnp
from jax import lax
from jax.experimental import pallas as pl
from jax.experimental.pallas import tpu as pltpu
from jax.experimental.pallas import tpu_sc as plsc

N_USERS = 1000000
N_ITEMS = 100000
N_FEATS = 100
K = 32
B = 4096

NC = 2
NS = 16
NW = NC * NS
BPW = B // NW

B_TILE = 256
D = K + 1
NJ = 2 + N_FEATS


_CHUNK = 16


def _gather_one_table(tab_hbm, idx_v, rows_v, win_v, sem):
  wvec = lax.iota(jnp.int32, 16) * D
  for c in range(BPW // _CHUNK):
    ch = idx_v[pl.ds(c * _CHUNK, 16)]
    for t in range(16):
      acol = pl.multiple_of((ch[t] >> 7) << 7, 128)
      pltpu.make_async_copy(tab_hbm.at[:, pl.ds(acol, 128)],
                            win_v.at[pl.ds(t * D, D)], sem).start()
    for t in range(16):
      pltpu.make_async_copy(tab_hbm.at[:, pl.ds(0, 128)],
                            win_v.at[pl.ds(t * D, D)], sem).wait()
    lvec = ch & 127
    for k in range(D):
      vals = plsc.load_gather(win_v, [wvec + k, lvec])
      rows_v[k, pl.ds(c * _CHUNK, 16)] = vals


def _sc_gather_body(u_hbm, i_hbm, ut_hbm, it_hbm, out_u, out_i,
                    uidx_v, iidx_v, urows_v, irows_v, win_v, sem):
  wid = lax.axis_index("s") * NC + lax.axis_index("c")
  base = wid * BPW
  pltpu.sync_copy(u_hbm.at[pl.ds(base, BPW)], uidx_v)
  pltpu.sync_copy(i_hbm.at[pl.ds(base, BPW)], iidx_v)

  _gather_one_table(ut_hbm, uidx_v, urows_v, win_v, sem)
  pltpu.sync_copy(urows_v, out_u.at[:, pl.ds(base, BPW)])
  _gather_one_table(it_hbm, iidx_v, irows_v, win_v, sem)
  pltpu.sync_copy(irows_v, out_i.at[:, pl.ds(base, BPW)])


@functools.cache
def _sc_gather():
  return pl.kernel(
      _sc_gather_body,
      mesh=plsc.VectorSubcoreMesh(core_axis_name="c", subcore_axis_name="s"),
      out_type=(
          jax.ShapeDtypeStruct((D, B), jnp.float32),
          jax.ShapeDtypeStruct((D, B), jnp.float32),
      ),
      scratch_types=[
          pltpu.VMEM((BPW,), jnp.int32),
          pltpu.VMEM((BPW,), jnp.int32),
          pltpu.VMEM((D, BPW), jnp.float32),
          pltpu.VMEM((D, BPW), jnp.float32),
          pltpu.VMEM((_CHUNK * D, 128), jnp.float32),
          pltpu.SemaphoreType.DMA,
      ],
      compiler_params=pltpu.CompilerParams(needs_layout_passes=False),
  )


def _tc_body(w0_ref, feats_ref, uv_ref, iv_ref, fkt_ref, f3l_ref, fwl_ref,
             s_ref, w_ref, v_ref):
  featsb = feats_ref[...]
  uvb = uv_ref[...]
  ivb = iv_ref[...]
  uvk = uvb[:K]
  ivk = ivb[:K]

  v_ref[0:1] = uvk[None]
  v_ref[1:2] = ivk[None]
  v_ref[2:] = featsb[:, None, :] * f3l_ref[...]

  wfeat = featsb * fwl_ref[...]
  w_ref[0:1] = uvb[K:]
  w_ref[1:2] = ivb[K:]
  w_ref[2:] = wfeat

  fk = fkt_ref[...]
  sv = uvk + ivk + jnp.dot(fk, featsb, preferred_element_type=jnp.float32)
  sq = (uvk * uvk + ivk * ivk
        + jnp.dot(fk * fk, featsb * featsb,
                  preferred_element_type=jnp.float32))
  v_ = 0.5 * (jnp.sum(sv * sv, axis=0, keepdims=True)
              - jnp.sum(sq, axis=0, keepdims=True))
  w_sum = uvb[K:] + ivb[K:] + jnp.sum(wfeat, axis=0, keepdims=True)
  s_ref[...] = w0_ref[0, 0] + w_sum + v_


def _tc_fm(w0, feats_t, uvt, ivt, fkt, f3l, fwl):
  grid = (B // B_TILE,)
  return pl.pallas_call(
      _tc_body,
      grid=grid,
      in_specs=[
          pl.BlockSpec(memory_space=pltpu.SMEM),
          pl.BlockSpec((N_FEATS, B_TILE), lambda b: (0, b)),
          pl.BlockSpec((D, B_TILE), lambda b: (0, b)),
          pl.BlockSpec((D, B_TILE), lambda b: (0, b)),
          pl.BlockSpec((K, N_FEATS), lambda b: (0, 0)),
          pl.BlockSpec((N_FEATS, K, B_TILE), lambda b: (0, 0, 0)),
          pl.BlockSpec((N_FEATS, B_TILE), lambda b: (0, 0)),
      ],
      out_specs=[
          pl.BlockSpec((1, B_TILE), lambda b: (0, b)),
          pl.BlockSpec((NJ, B_TILE), lambda b: (0, b)),
          pl.BlockSpec((NJ, K, B_TILE), lambda b: (0, 0, b)),
      ],
      out_shape=(
          jax.ShapeDtypeStruct((1, B), jnp.float32),
          jax.ShapeDtypeStruct((NJ, B), jnp.float32),
          jax.ShapeDtypeStruct((NJ, K, B), jnp.float32),
      ),
  )(w0, feats_t, uvt, ivt, fkt, f3l, fwl)


def kernel(u, i, feats, user_table, item_table, feat_table, w0):
  u1 = u.reshape(B).astype(jnp.int32)
  i1 = i.reshape(B).astype(jnp.int32)

  ut_t = user_table.T
  it_t = item_table.T
  feats_t = feats.T

  uvt, ivt = _sc_gather()(u1, i1, ut_t, it_t)

  ftab_k = feat_table[:, :K]
  fkt = ftab_k.T
  f3l = jnp.broadcast_to(ftab_k[:, :, None], (N_FEATS, K, B_TILE))
  fwl = jnp.broadcast_to(feat_table[:, K][:, None], (N_FEATS, B_TILE))

  s1, wt, vt = _tc_fm(w0.reshape(1, 1), feats_t, uvt, ivt, fkt, f3l, fwl)
  return (s1.reshape(B), wt.T, jnp.transpose(vt, (2, 0, 1)))


# speedup vs baseline: 5.1460x; 1.0481x over previous
import functools

import jax
import jax.numpy as jnp
from jax import lax
from jax.experimental import pallas as pl
from jax.experimental.pallas import tpu as pltpu
from jax.experimental.pallas import tpu_sc as plsc

N_USERS = 1000000
N_ITEMS = 100000
N_FEATS = 100
K = 32
B = 4096

NC = 2
NS = 16
NW = NC * NS
BPW = B // NW

B_TILE = 256
D = K + 1
NJ = 2 + N_FEATS


_CHUNK = 16


def _gather_one_table(tab_hbm, idx_v, rows_v, win_v, sem):
  wvec = lax.iota(jnp.int32, 16) * D
  for c in range(BPW // _CHUNK):
    ch = idx_v[pl.ds(c * _CHUNK, 16)]
    for t in range(16):
      acol = pl.multiple_of((ch[t] >> 7) << 7, 128)
      pltpu.make_async_copy(tab_hbm.at[:, pl.ds(acol, 128)],
                            win_v.at[pl.ds(t * D, D)], sem).start()
    for t in range(16):
      pltpu.make_async_copy(tab_hbm.at[:, pl.ds(0, 128)],
                            win_v.at[pl.ds(t * D, D)], sem).wait()
    lvec = ch & 127
    for k in range(D):
      vals = plsc.load_gather(win_v, [wvec + k, lvec])
      rows_v[k, pl.ds(c * _CHUNK, 16)] = vals


def _sc_gather_body(u_hbm, i_hbm, ut_hbm, it_hbm, out_u, out_i,
                    uidx_v, iidx_v, urows_v, irows_v, win_v, sem):
  wid = lax.axis_index("s") * NC + lax.axis_index("c")
  base = wid * BPW
  pltpu.sync_copy(u_hbm.at[pl.ds(base, BPW)], uidx_v)
  pltpu.sync_copy(i_hbm.at[pl.ds(base, BPW)], iidx_v)

  _gather_one_table(ut_hbm, uidx_v, urows_v, win_v, sem)
  pltpu.sync_copy(urows_v, out_u.at[:, pl.ds(base, BPW)])
  _gather_one_table(it_hbm, iidx_v, irows_v, win_v, sem)
  pltpu.sync_copy(irows_v, out_i.at[:, pl.ds(base, BPW)])


@functools.cache
def _sc_gather():
  return pl.kernel(
      _sc_gather_body,
      mesh=plsc.VectorSubcoreMesh(core_axis_name="c", subcore_axis_name="s"),
      out_type=(
          jax.ShapeDtypeStruct((D, B), jnp.float32),
          jax.ShapeDtypeStruct((D, B), jnp.float32),
      ),
      scratch_types=[
          pltpu.VMEM((BPW,), jnp.int32),
          pltpu.VMEM((BPW,), jnp.int32),
          pltpu.VMEM((D, BPW), jnp.float32),
          pltpu.VMEM((D, BPW), jnp.float32),
          pltpu.VMEM((_CHUNK * D, 128), jnp.float32),
          pltpu.SemaphoreType.DMA,
      ],
      compiler_params=pltpu.CompilerParams(needs_layout_passes=False),
  )


def _tc_a_body(feats_ref, fkt_ref, f3l_ref, fwl_ref,
               w_ref, v_ref, svf_ref, sqf_ref, wsf_ref):
  featsb = feats_ref[...]
  v_ref[2:] = featsb[:, None, :] * f3l_ref[...]
  wfeat = featsb * fwl_ref[...]
  w_ref[...] = wfeat
  fk = fkt_ref[...]
  svf_ref[...] = jnp.dot(fk, featsb, preferred_element_type=jnp.float32)
  sqf_ref[...] = jnp.dot(fk * fk, featsb * featsb,
                         preferred_element_type=jnp.float32)
  wsf_ref[...] = jnp.sum(wfeat, axis=0, keepdims=True)


def _tc_a(feats_t, fkt, f3l, fwl):
  grid = (B // B_TILE,)
  return pl.pallas_call(
      _tc_a_body,
      grid=grid,
      in_specs=[
          pl.BlockSpec((N_FEATS, B_TILE), lambda b: (0, b)),
          pl.BlockSpec((K, N_FEATS), lambda b: (0, 0)),
          pl.BlockSpec((N_FEATS, K, B_TILE), lambda b: (0, 0, 0)),
          pl.BlockSpec((N_FEATS, B_TILE), lambda b: (0, 0)),
      ],
      out_specs=[
          pl.BlockSpec((N_FEATS, B_TILE), lambda b: (0, b)),
          pl.BlockSpec((NJ, K, B_TILE), lambda b: (0, 0, b)),
          pl.BlockSpec((K, B_TILE), lambda b: (0, b)),
          pl.BlockSpec((K, B_TILE), lambda b: (0, b)),
          pl.BlockSpec((1, B_TILE), lambda b: (0, b)),
      ],
      out_shape=(
          jax.ShapeDtypeStruct((N_FEATS, B), jnp.float32),
          jax.ShapeDtypeStruct((NJ, K, B), jnp.float32),
          jax.ShapeDtypeStruct((K, B), jnp.float32),
          jax.ShapeDtypeStruct((K, B), jnp.float32),
          jax.ShapeDtypeStruct((1, B), jnp.float32),
      ),
  )(feats_t, fkt, f3l, fwl)


def _tc_b_body(w0_ref, uv_ref, iv_ref, svf_ref, sqf_ref, wsf_ref,
               v_al_ref, s_ref, w_ref, v_ref):
  del v_al_ref
  uvb = uv_ref[...]
  ivb = iv_ref[...]
  uvk = uvb[:K]
  ivk = ivb[:K]
  v_ref[0:1] = uvk[None]
  v_ref[1:2] = ivk[None]
  w_ref[0:1] = uvb[K:]
  w_ref[1:2] = ivb[K:]
  sv = uvk + ivk + svf_ref[...]
  sq = uvk * uvk + ivk * ivk + sqf_ref[...]
  v_ = 0.5 * (jnp.sum(sv * sv, axis=0, keepdims=True)
              - jnp.sum(sq, axis=0, keepdims=True))
  w_sum = uvb[K:] + ivb[K:] + wsf_ref[...]
  s_ref[...] = w0_ref[0, 0] + w_sum + v_


def _tc_b(w0, uvt, ivt, svf, sqf, wsf, va):
  grid = (B // B_TILE,)
  return pl.pallas_call(
      _tc_b_body,
      grid=grid,
      in_specs=[
          pl.BlockSpec(memory_space=pltpu.SMEM),
          pl.BlockSpec((D, B_TILE), lambda b: (0, b)),
          pl.BlockSpec((D, B_TILE), lambda b: (0, b)),
          pl.BlockSpec((K, B_TILE), lambda b: (0, b)),
          pl.BlockSpec((K, B_TILE), lambda b: (0, b)),
          pl.BlockSpec((1, B_TILE), lambda b: (0, b)),
          pl.BlockSpec(memory_space=pl.ANY),
      ],
      out_specs=[
          pl.BlockSpec((1, B_TILE), lambda b: (0, b)),
          pl.BlockSpec((2, B_TILE), lambda b: (0, b)),
          pl.BlockSpec((2, K, B_TILE), lambda b: (0, 0, b)),
      ],
      out_shape=(
          jax.ShapeDtypeStruct((1, B), jnp.float32),
          jax.ShapeDtypeStruct((2, B), jnp.float32),
          jax.ShapeDtypeStruct((NJ, K, B), jnp.float32),
      ),
      input_output_aliases={6: 2},
  )(w0, uvt, ivt, svf, sqf, wsf, va)


def kernel(u, i, feats, user_table, item_table, feat_table, w0):
  u1 = u.reshape(B).astype(jnp.int32)
  i1 = i.reshape(B).astype(jnp.int32)

  ut_t = user_table.T
  it_t = item_table.T
  feats_t = feats.T

  uvt, ivt = _sc_gather()(u1, i1, ut_t, it_t)

  ftab_k = feat_table[:, :K]
  fkt = ftab_k.T
  f3l = jnp.broadcast_to(ftab_k[:, :, None], (N_FEATS, K, B_TILE))
  fwl = jnp.broadcast_to(feat_table[:, K][:, None], (N_FEATS, B_TILE))

  wfeat, va, svf, sqf, wsf = _tc_a(feats_t, fkt, f3l, fwl)
  s1, wtop, vt = _tc_b(w0.reshape(1, 1), uvt, ivt, svf, sqf, wsf, va)
  wt = jnp.concatenate([wtop, wfeat], axis=0)
  return (s1.reshape(B), wt.T, jnp.transpose(vt, (2, 0, 1)))
